# baseline (speedup 1.0000x reference)
import functools

import jax
import jax.numpy as jnp
from jax import lax
from jax.experimental import pallas as pl
from jax.experimental.pallas import tpu as pltpu
from jax.experimental.pallas import tpu_sc as plsc

BATCH = 16384
HIST = 20
DIM = 32
TOTAL = BATCH * HIST

NC = 2
NS = 16
NW = NC * NS
BPW = TOTAL // NW

CHUNK = 128
K = 4
GROUP = CHUNK * K
NBUF = 4
WIN = 2
NGROUP = BPW // GROUP

_mesh = plsc.VectorSubcoreMesh(core_axis_name="c", subcore_axis_name="s")


@functools.partial(
    pl.kernel,
    mesh=_mesh,
    out_type=jax.ShapeDtypeStruct((TOTAL, DIM), jnp.float32),
    scratch_types=(
        [
            pltpu.VMEM((NGROUP * K, CHUNK), jnp.int32),
            pltpu.VMEM((NBUF, GROUP, DIM), jnp.float32),
        ]
        + [pltpu.SemaphoreType.DMA] * (2 * NBUF)
    ),
    compiler_params=pltpu.CompilerParams(use_tc_tiling_on_sc=False),
)
def _gather(idx_hbm, table_hbm, out_hbm, idx_v, rows_v, *sems):
    wid = lax.axis_index("s") * NC + lax.axis_index("c")
    pltpu.sync_copy(idx_hbm.at[wid], idx_v)

    gsem = sems[:NBUF]
    wsem = sems[NBUF:]
    gh = [None] * NBUF
    wh = [None] * NBUF

    def fire_group(g):
        buf = g % NBUF
        if wh[buf] is not None:
            wh[buf].wait()
            wh[buf] = None
        gh[buf] = [
            pltpu.async_copy(
                table_hbm.at[idx_v.at[g * K + b]],
                rows_v.at[buf].at[pl.ds(b * CHUNK, CHUNK)],
                gsem[buf],
            )
            for b in range(K)
        ]

    for g in range(min(WIN, NGROUP)):
        fire_group(g)
    for g in range(NGROUP):
        buf = g % NBUF
        for h in gh[buf]:
            h.wait()
        row0 = wid * BPW + g * GROUP
        wh[buf] = pltpu.async_copy(
            rows_v.at[buf], out_hbm.at[pl.ds(row0, GROUP)], wsem[buf]
        )
        if g + WIN < NGROUP:
            fire_group(g + WIN)
    for buf in range(NBUF):
        if wh[buf] is not None:
            wh[buf].wait()


def kernel(xs, table):
    idx = xs.astype(jnp.int32).reshape(NW, NGROUP * K, CHUNK)
    out = _gather(idx, table)
    return out.reshape(BATCH, HIST, DIM)
